# baseline (speedup 1.0000x reference)
import jax
import jax.numpy as jnp
from jax.experimental import pallas as pl
from jax.experimental.pallas import tpu as pltpu

_GS = 64


def _norm_kernel(idx_ref, x_ref, o_ref, onehot_t_ref):
    ng = onehot_t_ref.shape[0]
    n = onehot_t_ref.shape[1]

    @pl.when(pl.program_id(0) == 0)
    def _build_onehot():
        cc = 512
        gsmall = (
            jax.lax.broadcasted_iota(jnp.int32, (cc // _GS, cc), 1) // _GS
            == jax.lax.broadcasted_iota(jnp.int32, (cc // _GS, cc), 0)
        ).astype(jnp.bfloat16)
        civals = jax.lax.broadcasted_iota(jnp.int32, (cc, n), 1)
        for ci in range(n // cc):
            idx_chunk = idx_ref[pl.ds(ci * cc, cc), :]
            cmp = (idx_chunk == civals).astype(jnp.bfloat16)
            onehot_t_ref[pl.ds(ci * (cc // _GS), cc // _GS), :] = (
                jax.lax.dot_general(
                    gsmall, cmp, (((1,), (0,)), ((), ())),
                    preferred_element_type=jnp.float32,
                ).astype(jnp.bfloat16)
            )

    x = x_ref[...]
    onehot_t = onehot_t_ref[...]
    denom = jax.lax.dot_general(
        x.astype(jnp.bfloat16), onehot_t, (((1,), (1,)), ((), ())),
        preferred_element_type=jnp.float32,
    )
    recip = (1.0 / denom).astype(jnp.bfloat16)
    rexp = jax.lax.dot_general(
        recip, onehot_t, (((1,), (0,)), ((), ())),
        preferred_element_type=jnp.float32,
    )
    o_ref[...] = x * rexp


@jax.jit
def kernel(x, index):
    b, n = x.shape
    ng = n // _GS
    rows = 256
    idx2 = index.reshape(n, 1)
    return pl.pallas_call(
        _norm_kernel,
        grid=(b // rows,),
        in_specs=[
            pl.BlockSpec((n, 1), lambda i: (0, 0)),
            pl.BlockSpec((rows, n), lambda i: (i, 0)),
        ],
        out_specs=pl.BlockSpec((rows, n), lambda i: (i, 0)),
        out_shape=jax.ShapeDtypeStruct((b, n), x.dtype),
        scratch_shapes=[pltpu.VMEM((ng, n), jnp.bfloat16)],
    )(idx2, x)
